# baseline (speedup 1.0000x reference)
import functools

import jax
import jax.numpy as jnp
from jax import lax
from jax.experimental import pallas as pl
from jax.experimental.pallas import tpu as pltpu
from jax.experimental.pallas import tpu_sc as plsc

_B, _C, _H, _W = 4, 96, 384, 384
_N = 8192
_HW = _H * _W
_NC = 2
_NS = 16
_NW = _NC * _NS
_RW = 384
_NR = _B * _HW // 4
_S_TOTAL = _B * _N
_SPW = _S_TOTAL // _NW
_K = 32
_NCHUNK = _SPW // _K
_G = _K // 16


def _build_sc_call():
    mesh = plsc.VectorSubcoreMesh(core_axis_name="c", subcore_axis_name="s")

    @functools.partial(
        pl.kernel,
        mesh=mesh,
        compiler_params=pltpu.CompilerParams(needs_layout_passes=False),
        out_type=jax.ShapeDtypeStruct((_B * _C, _N), jnp.float32),
        scratch_types=[
            pltpu.VMEM((_SPW,), jnp.float32),
            pltpu.VMEM((_SPW,), jnp.float32),
            pltpu.VMEM((8, 16), jnp.float32),
            pltpu.VMEM((2, 4, _K), jnp.int32),
            pltpu.VMEM((2, 4, _G, 16), jnp.float32),
            pltpu.VMEM((2, 4, _G, 16), jnp.int32),
            pltpu.VMEM((2, 4, _K, _RW), jnp.float32),
            pltpu.VMEM((_C, 128), jnp.float32),
            pltpu.SemaphoreType.DMA,
            pltpu.SemaphoreType.DMA,
        ],
    )
    def retina(xT_hbm, tx_hbm, ty_hbm, f_hbm, out_hbm,
               tx_v, ty_v, fix_v, idx_v, w_v, cb_v, rows_v, out_v,
               semA, semB):
        wid = lax.axis_index("s") * _NC + lax.axis_index("c")
        base_s = wid * _SPW
        b = base_s // _N
        n0 = base_s % _N

        pltpu.sync_copy(tx_hbm.at[pl.ds(n0, _SPW)], tx_v)
        pltpu.sync_copy(ty_hbm.at[pl.ds(n0, _SPW)], ty_v)
        pltpu.sync_copy(f_hbm, fix_v)
        fxv = fix_v[2 * b, :]
        fyv = fix_v[2 * b + 1, :]
        bbase = b * _HW

        def geom_store(ci, g, slot):
            src = pl.ds(pl.multiple_of(ci * _K, _K) + g * 16, 16)
            gx = tx_v[src] + fxv
            gy = ty_v[src] + fyv
            ix = ((gx + 1.0) * _W - 1.0) * 0.5
            iy = ((gy + 1.0) * _H - 1.0) * 0.5
            ti = ix.astype(jnp.int32)
            tf = ti.astype(jnp.float32)
            ix0f = jnp.where(tf > ix, tf - 1.0, tf)
            ti = iy.astype(jnp.int32)
            tf = ti.astype(jnp.float32)
            iy0f = jnp.where(tf > iy, tf - 1.0, tf)
            wx1 = ix - ix0f
            wx0 = 1.0 - wx1
            wy1 = iy - iy0f
            wy0 = 1.0 - wy1
            ix0 = ix0f.astype(jnp.int32)
            ix1 = ix0 + 1
            iy0 = iy0f.astype(jnp.int32)
            iy1 = iy0 + 1
            vx0 = jnp.where((ix0 >= 0) & (ix0 <= _W - 1), 1.0, 0.0)
            vx1 = jnp.where((ix1 >= 0) & (ix1 <= _W - 1), 1.0, 0.0)
            vy0 = jnp.where((iy0 >= 0) & (iy0 <= _H - 1), 1.0, 0.0)
            vy1 = jnp.where((iy1 >= 0) & (iy1 <= _H - 1), 1.0, 0.0)
            cx0 = jnp.clip(ix0, 0, _W - 1)
            cx1 = jnp.clip(ix1, 0, _W - 1)
            cy0 = jnp.clip(iy0, 0, _H - 1)
            cy1 = jnp.clip(iy1, 0, _H - 1)
            pix = (bbase + cy0 * _W + cx0, bbase + cy0 * _W + cx1,
                   bbase + cy1 * _W + cx0, bbase + cy1 * _W + cx1)
            wgt = (wx0 * wy0 * vx0 * vy0, wx1 * wy0 * vx1 * vy0,
                   wx0 * wy1 * vx0 * vy1, wx1 * wy1 * vx1 * vy1)
            dst = pl.ds(g * 16, 16)
            for j in range(4):
                idx_v[slot, j, dst] = pix[j] >> 2
                cb_v[slot, j, g, :] = (pix[j] & 3) * _C
                w_v[slot, j, g, :] = wgt[j]

        def fire(ci, slot, sem):
            def geom_body(g, carry):
                geom_store(ci, g, slot)
                return carry
            lax.fori_loop(0, _G, geom_body, 0)
            for j in range(4):
                pltpu.async_copy(xT_hbm.at[idx_v.at[slot, j]],
                                 rows_v.at[slot, j], sem)

        def blend_store(ci, slot):
            sfull = jnp.full((16,), slot, jnp.int32)
            obase = pl.multiple_of((ci & 3) * _K, _K)
            def group_body(g, carry):
                kv = g * 16 + lax.iota(jnp.int32, 16)
                w0 = w_v[slot, 0, g, :]
                w1 = w_v[slot, 1, g, :]
                w2 = w_v[slot, 2, g, :]
                w3 = w_v[slot, 3, g, :]
                c0 = cb_v[slot, 0, g, :]
                c1 = cb_v[slot, 1, g, :]
                c2 = cb_v[slot, 2, g, :]
                c3 = cb_v[slot, 3, g, :]
                j0 = jnp.full((16,), 0, jnp.int32)
                j1 = jnp.full((16,), 1, jnp.int32)
                j2 = jnp.full((16,), 2, jnp.int32)
                j3 = jnp.full((16,), 3, jnp.int32)

                def chan_body(c4, ccarry):
                    cbase = c4 * 4
                    for s in range(4):
                        c = cbase + s
                        v0 = plsc.load_gather(rows_v, [sfull, j0, kv, c0 + c])
                        v1 = plsc.load_gather(rows_v, [sfull, j1, kv, c1 + c])
                        v2 = plsc.load_gather(rows_v, [sfull, j2, kv, c2 + c])
                        v3 = plsc.load_gather(rows_v, [sfull, j3, kv, c3 + c])
                        acc = w0 * v0 + w1 * v1 + w2 * v2 + w3 * v3
                        out_v[c, pl.ds(obase + g * 16, 16)] = acc
                    return ccarry

                lax.fori_loop(0, _C // 4, chan_body, 0)
                return carry
            lax.fori_loop(0, _G, group_body, 0)

            @pl.when((ci & 3) == 3)
            def _flush():
                off = pl.multiple_of(n0 + (ci >> 2) * 128, 128)
                pltpu.sync_copy(out_v,
                                out_hbm.at[pl.ds(b * _C, _C), pl.ds(off, 128)])

        def drain(slot, sem):
            for j in range(4):
                pltpu.make_async_copy(xT_hbm.at[idx_v.at[slot, j]],
                                      rows_v.at[slot, j], sem).wait()

        fire(0, 0, semA)

        def pair_body(pi, carry):
            c0 = 2 * pi
            c1 = c0 + 1
            c2 = jnp.minimum(c0 + 2, _NCHUNK - 1)
            fire(c1, 1, semB)
            drain(0, semA)
            blend_store(c0, 0)
            fire(c2, 0, semA)
            drain(1, semB)
            blend_store(c1, 1)
            return carry

        lax.fori_loop(0, _NCHUNK // 2, pair_body, 0)
        drain(0, semA)

    return retina


_sc_retina = _build_sc_call()


def kernel(x, fixations, tess):
    xT = jnp.transpose(x, (0, 2, 3, 1)).reshape(_NR, _RW)
    tx = tess[:, 0] + jnp.zeros((_N,), jnp.float32)
    ty = tess[:, 1] + jnp.zeros((_N,), jnp.float32)
    fpad = jnp.broadcast_to(fixations.reshape(8, 1), (8, 16)) + jnp.zeros(
        (8, 16), jnp.float32)
    out = _sc_retina(xT, tx, ty, fpad)
    return out.reshape(_B, _C, _N)


# speedup vs baseline: 1.3432x; 1.3432x over previous
import functools

import jax
import jax.numpy as jnp
from jax import lax
from jax.experimental import pallas as pl
from jax.experimental.pallas import tpu as pltpu
from jax.experimental.pallas import tpu_sc as plsc

_B, _C, _H, _W = 4, 96, 384, 384
_N = 8192
_HW = _H * _W
_NC = 2
_NS = 16
_NW = _NC * _NS
_CP = 128
_S_TOTAL = _B * _N
_SPW = _S_TOTAL // _NW
_K = 32
_NCHUNK = _SPW // _K
_G = _K // 16


def _build_sc_call():
    mesh = plsc.VectorSubcoreMesh(core_axis_name="c", subcore_axis_name="s")

    @functools.partial(
        pl.kernel,
        mesh=mesh,
        compiler_params=pltpu.CompilerParams(needs_layout_passes=False),
        out_type=jax.ShapeDtypeStruct((_B * _C, _N), jnp.float32),
        scratch_types=[
            pltpu.VMEM((_SPW,), jnp.float32),
            pltpu.VMEM((_SPW,), jnp.float32),
            pltpu.VMEM((8, 16), jnp.float32),
            pltpu.VMEM((2, 4, _K), jnp.int32),
            pltpu.VMEM((2, 4, _G, 16), jnp.float32),
            pltpu.VMEM((2, 4, _K, _CP), jnp.float32),
            pltpu.VMEM((_C, 128), jnp.float32),
            pltpu.SemaphoreType.DMA,
            pltpu.SemaphoreType.DMA,
        ],
    )
    def retina(xT_hbm, tx_hbm, ty_hbm, f_hbm, out_hbm,
               tx_v, ty_v, fix_v, idx_v, w_v, rows_v, out_v, semA, semB):
        wid = lax.axis_index("s") * _NC + lax.axis_index("c")
        base_s = wid * _SPW
        b = base_s // _N
        n0 = base_s % _N

        pltpu.sync_copy(tx_hbm.at[pl.ds(n0, _SPW)], tx_v)
        pltpu.sync_copy(ty_hbm.at[pl.ds(n0, _SPW)], ty_v)
        pltpu.sync_copy(f_hbm, fix_v)
        fxv = fix_v[2 * b, :]
        fyv = fix_v[2 * b + 1, :]
        bbase = b * _HW

        def geom_store(ci, g, slot):
            src = pl.ds(pl.multiple_of(ci * _K, _K) + g * 16, 16)
            gx = tx_v[src] + fxv
            gy = ty_v[src] + fyv
            ix = ((gx + 1.0) * _W - 1.0) * 0.5
            iy = ((gy + 1.0) * _H - 1.0) * 0.5
            ti = ix.astype(jnp.int32)
            tf = ti.astype(jnp.float32)
            ix0f = jnp.where(tf > ix, tf - 1.0, tf)
            ti = iy.astype(jnp.int32)
            tf = ti.astype(jnp.float32)
            iy0f = jnp.where(tf > iy, tf - 1.0, tf)
            wx1 = ix - ix0f
            wx0 = 1.0 - wx1
            wy1 = iy - iy0f
            wy0 = 1.0 - wy1
            ix0 = ix0f.astype(jnp.int32)
            ix1 = ix0 + 1
            iy0 = iy0f.astype(jnp.int32)
            iy1 = iy0 + 1
            vx0 = jnp.where((ix0 >= 0) & (ix0 <= _W - 1), 1.0, 0.0)
            vx1 = jnp.where((ix1 >= 0) & (ix1 <= _W - 1), 1.0, 0.0)
            vy0 = jnp.where((iy0 >= 0) & (iy0 <= _H - 1), 1.0, 0.0)
            vy1 = jnp.where((iy1 >= 0) & (iy1 <= _H - 1), 1.0, 0.0)
            cx0 = jnp.clip(ix0, 0, _W - 1)
            cx1 = jnp.clip(ix1, 0, _W - 1)
            cy0 = jnp.clip(iy0, 0, _H - 1)
            cy1 = jnp.clip(iy1, 0, _H - 1)
            dst = pl.ds(g * 16, 16)
            idx_v[slot, 0, dst] = bbase + cy0 * _W + cx0
            idx_v[slot, 1, dst] = bbase + cy0 * _W + cx1
            idx_v[slot, 2, dst] = bbase + cy1 * _W + cx0
            idx_v[slot, 3, dst] = bbase + cy1 * _W + cx1
            w_v[slot, 0, g, :] = wx0 * wy0 * vx0 * vy0
            w_v[slot, 1, g, :] = wx1 * wy0 * vx1 * vy0
            w_v[slot, 2, g, :] = wx0 * wy1 * vx0 * vy1
            w_v[slot, 3, g, :] = wx1 * wy1 * vx1 * vy1

        def fire(ci, slot, sem):
            def geom_body(g, carry):
                geom_store(ci, g, slot)
                return carry
            lax.fori_loop(0, _G, geom_body, 0)
            for j in range(4):
                pltpu.async_copy(xT_hbm.at[idx_v.at[slot, j]],
                                 rows_v.at[slot, j], sem)

        _ROWIDX = [jnp.arange(cc * 16, cc * 16 + 16, dtype=jnp.int32)
                   for cc in range(_C // 16)]

        def blend_store(ci, slot):
            ocol0 = (ci & 3) * _K
            for g in range(_G):
                w0r = w_v[slot, 0, g, :]
                w1r = w_v[slot, 1, g, :]
                w2r = w_v[slot, 2, g, :]
                w3r = w_v[slot, 3, g, :]
                for i in range(16):
                    k = g * 16 + i
                    w0 = jnp.full((16,), w0r[i], jnp.float32)
                    w1 = jnp.full((16,), w1r[i], jnp.float32)
                    w2 = jnp.full((16,), w2r[i], jnp.float32)
                    w3 = jnp.full((16,), w3r[i], jnp.float32)
                    colv = jnp.full((16,), ocol0 + k, jnp.int32)
                    for cc in range(_C // 16):
                        csl = pl.ds(cc * 16, 16)
                        acc = (w0 * rows_v[slot, 0, k, csl]
                               + w1 * rows_v[slot, 1, k, csl])
                        acc = acc + (w2 * rows_v[slot, 2, k, csl]
                                     + w3 * rows_v[slot, 3, k, csl])
                        plsc.store_scatter(out_v, [_ROWIDX[cc], colv], acc)

            @pl.when((ci & 3) == 3)
            def _flush():
                off = pl.multiple_of(n0 + (ci >> 2) * 128, 128)
                pltpu.sync_copy(out_v,
                                out_hbm.at[pl.ds(b * _C, _C), pl.ds(off, 128)])

        def drain(slot, sem):
            for j in range(4):
                pltpu.make_async_copy(xT_hbm.at[idx_v.at[slot, j]],
                                      rows_v.at[slot, j], sem).wait()

        fire(0, 0, semA)

        def pair_body(pi, carry):
            c0 = 2 * pi
            c1 = c0 + 1
            c2 = jnp.minimum(c0 + 2, _NCHUNK - 1)
            fire(c1, 1, semB)
            drain(0, semA)
            blend_store(c0, 0)
            fire(c2, 0, semA)
            drain(1, semB)
            blend_store(c1, 1)
            return carry

        lax.fori_loop(0, _NCHUNK // 2, pair_body, 0)
        drain(0, semA)

    return retina


_sc_retina = _build_sc_call()


def kernel(x, fixations, tess):
    xT = jnp.pad(jnp.transpose(x, (0, 2, 3, 1)).reshape(_B * _HW, _C),
                 ((0, 0), (0, _CP - _C)))
    tx = tess[:, 0] + jnp.zeros((_N,), jnp.float32)
    ty = tess[:, 1] + jnp.zeros((_N,), jnp.float32)
    fpad = jnp.broadcast_to(fixations.reshape(8, 1), (8, 16)) + jnp.zeros(
        (8, 16), jnp.float32)
    out = _sc_retina(xT, tx, ty, fpad)
    return out.reshape(_B, _C, _N)


# speedup vs baseline: 1.4134x; 1.0522x over previous
import functools

import jax
import jax.numpy as jnp
from jax import lax
from jax.experimental import pallas as pl
from jax.experimental.pallas import tpu as pltpu
from jax.experimental.pallas import tpu_sc as plsc

_B, _C, _H, _W = 4, 96, 384, 384
_N = 8192
_HW = _H * _W
_NC = 2
_NS = 16
_NW = _NC * _NS
_CP = 128
_S_TOTAL = _B * _N
_SPW = _S_TOTAL // _NW
_K = 64
_NCHUNK = _SPW // _K
_G = _K // 16


def _build_sc_call():
    mesh = plsc.VectorSubcoreMesh(core_axis_name="c", subcore_axis_name="s")

    @functools.partial(
        pl.kernel,
        mesh=mesh,
        out_type=jax.ShapeDtypeStruct((_S_TOTAL, _C), jnp.float32),
        scratch_types=[
            pltpu.VMEM((_SPW,), jnp.float32),
            pltpu.VMEM((_SPW,), jnp.float32),
            pltpu.VMEM((8, 16), jnp.float32),
            pltpu.VMEM((2, 4, _K), jnp.int32),
            pltpu.VMEM((2, 4, _G, 16), jnp.float32),
            pltpu.VMEM((2, 4, _K, _CP), jnp.float32),
            pltpu.VMEM((_K, _C), jnp.float32),
            pltpu.SemaphoreType.DMA,
            pltpu.SemaphoreType.DMA,
        ],
    )
    def retina(xT_hbm, tx_hbm, ty_hbm, f_hbm, out_hbm,
               tx_v, ty_v, fix_v, idx_v, w_v, rows_v, out_v, semA, semB):
        wid = lax.axis_index("s") * _NC + lax.axis_index("c")
        base_s = wid * _SPW
        b = base_s // _N
        n0 = base_s % _N

        pltpu.sync_copy(tx_hbm.at[pl.ds(n0, _SPW)], tx_v)
        pltpu.sync_copy(ty_hbm.at[pl.ds(n0, _SPW)], ty_v)
        pltpu.sync_copy(f_hbm, fix_v)
        fxv = fix_v[2 * b, :]
        fyv = fix_v[2 * b + 1, :]
        bbase = b * _HW

        def geom_store(ci, g, slot):
            src = pl.ds(pl.multiple_of(ci * _K, _K) + g * 16, 16)
            gx = tx_v[src] + fxv
            gy = ty_v[src] + fyv
            ix = ((gx + 1.0) * _W - 1.0) * 0.5
            iy = ((gy + 1.0) * _H - 1.0) * 0.5
            ti = ix.astype(jnp.int32)
            tf = ti.astype(jnp.float32)
            ix0f = jnp.where(tf > ix, tf - 1.0, tf)
            ti = iy.astype(jnp.int32)
            tf = ti.astype(jnp.float32)
            iy0f = jnp.where(tf > iy, tf - 1.0, tf)
            wx1 = ix - ix0f
            wx0 = 1.0 - wx1
            wy1 = iy - iy0f
            wy0 = 1.0 - wy1
            ix0 = ix0f.astype(jnp.int32)
            ix1 = ix0 + 1
            iy0 = iy0f.astype(jnp.int32)
            iy1 = iy0 + 1
            vx0 = jnp.where((ix0 >= 0) & (ix0 <= _W - 1), 1.0, 0.0)
            vx1 = jnp.where((ix1 >= 0) & (ix1 <= _W - 1), 1.0, 0.0)
            vy0 = jnp.where((iy0 >= 0) & (iy0 <= _H - 1), 1.0, 0.0)
            vy1 = jnp.where((iy1 >= 0) & (iy1 <= _H - 1), 1.0, 0.0)
            cx0 = jnp.clip(ix0, 0, _W - 1)
            cx1 = jnp.clip(ix1, 0, _W - 1)
            cy0 = jnp.clip(iy0, 0, _H - 1)
            cy1 = jnp.clip(iy1, 0, _H - 1)
            dst = pl.ds(g * 16, 16)
            idx_v[slot, 0, dst] = bbase + cy0 * _W + cx0
            idx_v[slot, 1, dst] = bbase + cy0 * _W + cx1
            idx_v[slot, 2, dst] = bbase + cy1 * _W + cx0
            idx_v[slot, 3, dst] = bbase + cy1 * _W + cx1
            w_v[slot, 0, g, :] = wx0 * wy0 * vx0 * vy0
            w_v[slot, 1, g, :] = wx1 * wy0 * vx1 * vy0
            w_v[slot, 2, g, :] = wx0 * wy1 * vx0 * vy1
            w_v[slot, 3, g, :] = wx1 * wy1 * vx1 * vy1

        def fire(ci, slot, sem):
            def geom_body(g, carry):
                geom_store(ci, g, slot)
                return carry
            lax.fori_loop(0, _G, geom_body, 0)
            for j in range(4):
                pltpu.async_copy(xT_hbm.at[idx_v.at[slot, j]],
                                 rows_v.at[slot, j], sem)

        def blend_store(ci, slot):
            def group_body(g, carry):
                w0r = w_v[slot, 0, g, :]
                w1r = w_v[slot, 1, g, :]
                w2r = w_v[slot, 2, g, :]
                w3r = w_v[slot, 3, g, :]
                kbase = g * 16
                for i in range(16):
                    k = kbase + i
                    w0 = jnp.full((16,), w0r[i], jnp.float32)
                    w1 = jnp.full((16,), w1r[i], jnp.float32)
                    w2 = jnp.full((16,), w2r[i], jnp.float32)
                    w3 = jnp.full((16,), w3r[i], jnp.float32)
                    for cc in range(_C // 16):
                        csl = pl.ds(cc * 16, 16)
                        acc = w0 * rows_v[slot, 0, k, csl]
                        acc = acc + w1 * rows_v[slot, 1, k, csl]
                        acc = acc + w2 * rows_v[slot, 2, k, csl]
                        acc = acc + w3 * rows_v[slot, 3, k, csl]
                        out_v[k, csl] = acc
                return carry
            lax.fori_loop(0, _G, group_body, 0)
            pltpu.sync_copy(out_v, out_hbm.at[pl.ds(base_s + ci * _K, _K)])

        def drain(slot, sem):
            for j in range(4):
                pltpu.make_async_copy(xT_hbm.at[idx_v.at[slot, j]],
                                      rows_v.at[slot, j], sem).wait()

        fire(0, 0, semA)

        def pair_body(pi, carry):
            c0 = 2 * pi
            c1 = c0 + 1
            c2 = jnp.minimum(c0 + 2, _NCHUNK - 1)
            fire(c1, 1, semB)
            drain(0, semA)
            blend_store(c0, 0)
            fire(c2, 0, semA)
            drain(1, semB)
            blend_store(c1, 1)
            return carry

        lax.fori_loop(0, _NCHUNK // 2, pair_body, 0)
        drain(0, semA)

    return retina


_sc_retina = _build_sc_call()


def kernel(x, fixations, tess):
    xT = jnp.pad(jnp.transpose(x, (0, 2, 3, 1)).reshape(_B * _HW, _C),
                 ((0, 0), (0, _CP - _C)))
    tx = tess[:, 0] + jnp.zeros((_N,), jnp.float32)
    ty = tess[:, 1] + jnp.zeros((_N,), jnp.float32)
    fpad = jnp.broadcast_to(fixations.reshape(8, 1), (8, 16)) + jnp.zeros(
        (8, 16), jnp.float32)
    out = _sc_retina(xT, tx, ty, fpad)
    return jnp.transpose(out.reshape(_B, _N, _C), (0, 2, 1))
